# Initial kernel scaffold; baseline (speedup 1.0000x reference)
import jax
import jax.numpy as jnp
from jax.experimental import pallas as pl


def kernel(x, edge_index, edge_attr, W1, We1, as1, ad1, ae1, b1, W2, We2, as2, ad2, ae2, b2, Wf, bf):
    raise NotImplementedError("write your pallas kernel here")


# speedup vs baseline: 16.6076x; 16.6076x over previous
import jax
import jax.numpy as jnp
from jax import lax
from jax.experimental import pallas as pl
from jax.experimental.pallas import tpu as pltpu
from jax.experimental.pallas import tpu_sc as plsc

N = 10000
E = 320000
ETOT = E + N
C = 128
A = 16
CF = C // 2
NP = N + 16
DB = N + 16
DR = 157
NSEG = 10176
NC = 2
NS = 16
K = 112
G = 186
CH = G * K
EP = NS * CH
SL = NSEG // NS
GS = 31
NSLAB = G // GS

f32 = jnp.float32
i32 = jnp.int32


def _tc1_body(x_ref, W1_ref, as1_ref, ad1_ref, eat_ref, We1_ref, ae1_ref,
              We2_ref, ae2_ref,
              hlo_ref, hhi_ref, asrc_ref, adst_ref, ae1p_ref, ae2p_ref):
    h = jnp.dot(x_ref[...], W1_ref[...], preferred_element_type=f32)
    hlo_ref[...] = h[:, :CF]
    hhi_ref[...] = h[:, CF:]
    z16 = jnp.zeros((16,), f32)
    asrc_ref[...] = jnp.concatenate([jnp.dot(h, as1_ref[...]), z16])
    adst_ref[...] = jnp.concatenate([jnp.dot(h, ad1_ref[...]), z16])
    for We_ref, ae_ref, out_ref in ((We1_ref, ae1_ref, ae1p_ref),
                                    (We2_ref, ae2_ref, ae2p_ref)):
        v0 = jnp.sum(We_ref[0, :] * ae_ref[...])
        v1 = jnp.sum(We_ref[1, :] * ae_ref[...])
        ae_real = eat_ref[0, :] * v0 + eat_ref[1, :] * v1
        loopv = jnp.mean(eat_ref[0, :]) * v0 + jnp.mean(eat_ref[1, :]) * v1
        out_ref[...] = jnp.concatenate(
            [ae_real, jnp.full((EP - E,), loopv, f32)])


_tc1 = pl.pallas_call(
    _tc1_body,
    out_shape=[
        jax.ShapeDtypeStruct((N, CF), f32),
        jax.ShapeDtypeStruct((N, CF), f32),
        jax.ShapeDtypeStruct((NP,), f32),
        jax.ShapeDtypeStruct((NP,), f32),
        jax.ShapeDtypeStruct((EP,), f32),
        jax.ShapeDtypeStruct((EP,), f32),
    ],
)


def _norm_h(p_ref, den_ref, b_ref):
    acc = jnp.concatenate([p_ref[0, :N, :], p_ref[1, :N, :]], axis=-1)
    den = den_ref[...][:N]
    return jnp.maximum(acc / jnp.maximum(den, 1e-16)[:, None]
                       + b_ref[...][None, :], 0.0)


def _tc2_body(p_ref, den_ref, b_ref, W_ref, as_ref, ad_ref,
              hlo_ref, hhi_ref, asrc_ref, adst_ref):
    h = _norm_h(p_ref, den_ref, b_ref)
    hw = jnp.dot(h, W_ref[...], preferred_element_type=f32)
    hlo_ref[...] = hw[:, :CF]
    hhi_ref[...] = hw[:, CF:]
    z16 = jnp.zeros((16,), f32)
    asrc_ref[...] = jnp.concatenate([jnp.dot(hw, as_ref[...]), z16])
    adst_ref[...] = jnp.concatenate([jnp.dot(hw, ad_ref[...]), z16])


_tc2 = pl.pallas_call(
    _tc2_body,
    out_shape=[
        jax.ShapeDtypeStruct((N, CF), f32),
        jax.ShapeDtypeStruct((N, CF), f32),
        jax.ShapeDtypeStruct((NP,), f32),
        jax.ShapeDtypeStruct((NP,), f32),
    ],
)


def _tc3_body(p_ref, den_ref, b_ref, Wf_ref, bf_ref, y_ref):
    h = _norm_h(p_ref, den_ref, b_ref)
    g = jnp.mean(h, axis=0)
    y_ref[...] = jnp.tanh(jnp.dot(g, Wf_ref[...]) + bf_ref[...])


_tc3 = pl.pallas_call(
    _tc3_body,
    out_shape=jax.ShapeDtypeStruct((A,), f32),
)


def _sc_body(src_ref, dst_ref, ae_ref, asrc_ref, adst_ref, hlo_ref, hhi_ref,
             outp_ref,
             asrc_v, adst_v, src_v, dst_v, ae_v, w_v, drow_v, dcol_v,
             exbuf_v, rows_v, acc_s, sem):
    c = lax.axis_index("c")
    s = lax.axis_index("s")
    z16 = jnp.zeros((16,), f32)

    def _zr(r, carry):
        for j in range(CF // 16):
            rows_v[r, pl.ds(j * 16, 16)] = z16
            exbuf_v[r, pl.ds(j * 16, 16)] = z16
        return carry
    lax.fori_loop(0, K, _zr, 0)
    base = s * SL
    for t in range(5):
        pltpu.sync_copy(rows_v, acc_s.at[pl.ds(base + t * K, K)])
    pltpu.sync_copy(rows_v.at[pl.ds(0, SL - 5 * K)],
                    acc_s.at[pl.ds(base + 5 * K, SL - 5 * K)])
    plsc.subcore_barrier()

    pltpu.sync_copy(asrc_ref, asrc_v)
    pltpu.sync_copy(adst_ref, adst_v)

    iota16 = lax.iota(i32, 16)

    def _group(g, carry):
        def _alpha(k, carry2):
            si = src_v[g, pl.ds(k * 16, 16)]
            di = dst_v[g, pl.ds(k * 16, 16)]
            al = (plsc.load_gather(asrc_v, [si])
                  + plsc.load_gather(adst_v, [di])
                  + ae_v[g, pl.ds(k * 16, 16)])
            al = jnp.where(al >= 0, al, 0.2 * al)
            ex = jnp.exp(al)
            w_v[pl.ds(k * 16, 16)] = ex
            dcol = lax.bitwise_and(di, 63)
            drow_v[pl.ds(k * 16, 16)] = DB + lax.shift_right_logical(di, 6)
            dcol_v[pl.ds(k * 16, 16)] = dcol
            plsc.store_scatter(exbuf_v, [k * 16 + iota16, dcol], ex)
            return carry2
        lax.fori_loop(0, K // 16, _alpha, 0)

        @pl.when(c == 0)
        def _():
            pltpu.async_copy(hlo_ref.at[src_v.at[g]], rows_v, sem).wait()

        @pl.when(c == 1)
        def _():
            pltpu.async_copy(hhi_ref.at[src_v.at[g]], rows_v, sem).wait()

        def _scale(r, carry2):
            bc = plsc.load_gather(w_v, [jnp.full((16,), r, i32)])
            for j in range(CF // 16):
                rows_v[r, pl.ds(j * 16, 16)] = rows_v[r, pl.ds(j * 16, 16)] * bc
            return carry2
        lax.fori_loop(0, K, _scale, 0)

        pltpu.sync_copy(rows_v, acc_s.at[dst_v.at[g]], add=True)

        @pl.when(c == 0)
        def _():
            pltpu.sync_copy(exbuf_v, acc_s.at[drow_v], add=True)

        def _clean(k, carry2):
            plsc.store_scatter(
                exbuf_v, [k * 16 + iota16, dcol_v[pl.ds(k * 16, 16)]],
                jnp.zeros((16,), f32))
            return carry2
        lax.fori_loop(0, K // 16, _clean, 0)
        return carry

    def _slab(sl, carry):
        pltpu.sync_copy(src_ref.at[s, pl.ds(sl * GS, GS)], src_v)
        pltpu.sync_copy(dst_ref.at[s, pl.ds(sl * GS, GS)], dst_v)
        pltpu.sync_copy(ae_ref.at[s, pl.ds(sl * GS, GS)], ae_v)
        lax.fori_loop(0, GS, _group, 0)
        return carry
    lax.fori_loop(0, NSLAB, _slab, 0)

    plsc.subcore_barrier()
    for t in range(5):
        pltpu.sync_copy(acc_s.at[pl.ds(base + t * K, K)],
                        outp_ref.at[c, pl.ds(base + t * K, K)])
    pltpu.sync_copy(acc_s.at[pl.ds(base + 5 * K, SL - 5 * K)],
                    outp_ref.at[c, pl.ds(base + 5 * K, SL - 5 * K)])


_sc_layer = pl.kernel(
    _sc_body,
    out_type=jax.ShapeDtypeStruct((NC, NSEG, CF), f32),
    mesh=plsc.VectorSubcoreMesh(core_axis_name="c", subcore_axis_name="s"),
    compiler_params=pltpu.CompilerParams(needs_layout_passes=False,
                                         use_tc_tiling_on_sc=False),
    scratch_types=[
        pltpu.VMEM((NP,), f32),
        pltpu.VMEM((NP,), f32),
        pltpu.VMEM((GS, K), i32),
        pltpu.VMEM((GS, K), i32),
        pltpu.VMEM((GS, K), f32),
        pltpu.VMEM((K,), f32),
        pltpu.VMEM((K,), i32),
        pltpu.VMEM((K,), i32),
        pltpu.VMEM((K, CF), f32),
        pltpu.VMEM((K, CF), f32),
        pltpu.VMEM_SHARED((NSEG, CF), f32),
        pltpu.SemaphoreType.DMA,
    ],
)


@jax.jit
def kernel(x, edge_index, edge_attr, W1, We1, as1, ad1, ae1, b1,
           W2, We2, as2, ad2, ae2, b2, Wf, bf):
    loop = jnp.arange(N, dtype=i32)
    src = jnp.concatenate([edge_index[0].astype(i32), loop,
                           jnp.zeros((EP - ETOT,), i32)])
    dst = jnp.concatenate([edge_index[1].astype(i32), loop,
                           jnp.full((EP - ETOT,), N, i32)])
    src3 = src.reshape(NS, G, K)
    dst3 = dst.reshape(NS, G, K)
    eat = edge_attr.T

    hlo, hhi, asrc1, adst1, ae1p, ae2p = _tc1(x, W1, as1, ad1, eat, We1, ae1,
                                              We2, ae2)
    p1 = _sc_layer(src3, dst3, ae1p.reshape(NS, G, K), asrc1, adst1,
                   hlo, hhi)
    den1 = p1[0, DB:DB + DR, :].reshape(DR * CF)
    hlo2, hhi2, asrc2, adst2 = _tc2(p1, den1, b1, W2, as2, ad2)
    p2 = _sc_layer(src3, dst3, ae2p.reshape(NS, G, K), asrc2, adst2,
                   hlo2, hhi2)
    den2 = p2[0, DB:DB + DR, :].reshape(DR * CF)
    return _tc3(p2, den2, b2, Wf, bf)


# speedup vs baseline: 18.7660x; 1.1300x over previous
import jax
import jax.numpy as jnp
from jax import lax
from jax.experimental import pallas as pl
from jax.experimental.pallas import tpu as pltpu
from jax.experimental.pallas import tpu_sc as plsc

N = 10000
E = 320000
ETOT = E + N
C = 128
A = 16
CF = C // 2
NP = N + 16
DB = N + 16
DR = 157
NSEG = 10176
NC = 2
NS = 16
K = 112
G = 186
CH = G * K
EP = NS * CH
SL = NSEG // NS
GS = 31
NSLAB = G // GS

f32 = jnp.float32
i32 = jnp.int32


def _tc1_body(x_ref, W1_ref, as1_ref, ad1_ref, eat_ref, We1_ref, ae1_ref,
              We2_ref, ae2_ref,
              hlo_ref, hhi_ref, asrc_ref, adst_ref, ae1p_ref, ae2p_ref):
    h = jnp.dot(x_ref[...], W1_ref[...], preferred_element_type=f32)
    hlo_ref[...] = h[:, :CF]
    hhi_ref[...] = h[:, CF:]
    z16 = jnp.zeros((16,), f32)
    asrc_ref[...] = jnp.concatenate([jnp.dot(h, as1_ref[...]), z16])
    adst_ref[...] = jnp.concatenate([jnp.dot(h, ad1_ref[...]), z16])
    for We_ref, ae_ref, out_ref in ((We1_ref, ae1_ref, ae1p_ref),
                                    (We2_ref, ae2_ref, ae2p_ref)):
        v0 = jnp.sum(We_ref[0, :] * ae_ref[...])
        v1 = jnp.sum(We_ref[1, :] * ae_ref[...])
        ae_real = eat_ref[0, :] * v0 + eat_ref[1, :] * v1
        loopv = jnp.mean(eat_ref[0, :]) * v0 + jnp.mean(eat_ref[1, :]) * v1
        out_ref[...] = jnp.concatenate(
            [ae_real, jnp.full((EP - E,), loopv, f32)])


_tc1 = pl.pallas_call(
    _tc1_body,
    out_shape=[
        jax.ShapeDtypeStruct((N, CF), f32),
        jax.ShapeDtypeStruct((N, CF), f32),
        jax.ShapeDtypeStruct((NP,), f32),
        jax.ShapeDtypeStruct((NP,), f32),
        jax.ShapeDtypeStruct((EP,), f32),
        jax.ShapeDtypeStruct((EP,), f32),
    ],
)


def _norm_h(p_ref, den_ref, b_ref):
    acc = jnp.concatenate([p_ref[0, :N, :], p_ref[1, :N, :]], axis=-1)
    den = den_ref[...][:N]
    return jnp.maximum(acc / jnp.maximum(den, 1e-16)[:, None]
                       + b_ref[...][None, :], 0.0)


def _tc2_body(p_ref, den_ref, b_ref, W_ref, as_ref, ad_ref,
              hlo_ref, hhi_ref, asrc_ref, adst_ref):
    h = _norm_h(p_ref, den_ref, b_ref)
    hw = jnp.dot(h, W_ref[...], preferred_element_type=f32)
    hlo_ref[...] = hw[:, :CF]
    hhi_ref[...] = hw[:, CF:]
    z16 = jnp.zeros((16,), f32)
    asrc_ref[...] = jnp.concatenate([jnp.dot(hw, as_ref[...]), z16])
    adst_ref[...] = jnp.concatenate([jnp.dot(hw, ad_ref[...]), z16])


_tc2 = pl.pallas_call(
    _tc2_body,
    out_shape=[
        jax.ShapeDtypeStruct((N, CF), f32),
        jax.ShapeDtypeStruct((N, CF), f32),
        jax.ShapeDtypeStruct((NP,), f32),
        jax.ShapeDtypeStruct((NP,), f32),
    ],
)


def _tc3_body(p_ref, den_ref, b_ref, Wf_ref, bf_ref, y_ref):
    h = _norm_h(p_ref, den_ref, b_ref)
    g = jnp.mean(h, axis=0)
    y_ref[...] = jnp.tanh(jnp.dot(g, Wf_ref[...]) + bf_ref[...])


_tc3 = pl.pallas_call(
    _tc3_body,
    out_shape=jax.ShapeDtypeStruct((A,), f32),
)


def _sc_body(src_ref, dst_ref, ae_ref, asrc_ref, adst_ref, hlo_ref, hhi_ref,
             outp_ref,
             asrc_v, adst_v, src_v, dst_v, ae_v, w_sv, drow_sv, dcol_sv,
             in0, in1, out0, out1, exb0, exb1, acc_s,
             gsem0, gsem1, ssem0, ssem1):
    c = lax.axis_index("c")
    s = lax.axis_index("s")
    z16 = jnp.zeros((16,), f32)
    ins = (in0, in1)
    outs = (out0, out1)
    exbs = (exb0, exb1)
    gsems = (gsem0, gsem1)
    ssems = (ssem0, ssem1)

    def _zr(r, carry):
        for j in range(CF // 16):
            out0[r, pl.ds(j * 16, 16)] = z16
            exb0[r, pl.ds(j * 16, 16)] = z16
            exb1[r, pl.ds(j * 16, 16)] = z16
        return carry
    lax.fori_loop(0, K, _zr, 0)
    base = s * SL
    for t in range(5):
        pltpu.sync_copy(out0, acc_s.at[pl.ds(base + t * K, K)])
    pltpu.sync_copy(out0.at[pl.ds(0, SL - 5 * K)],
                    acc_s.at[pl.ds(base + 5 * K, SL - 5 * K)])
    plsc.subcore_barrier()

    pltpu.sync_copy(asrc_ref, asrc_v)
    pltpu.sync_copy(adst_ref, adst_v)

    iota16 = lax.iota(i32, 16)

    def _drain(sem, buf):
        pltpu.make_async_copy(hlo_ref.at[pl.ds(0, K)], buf, sem).wait()

    def _gstart(g, b):
        @pl.when(c == 0)
        def _():
            pltpu.async_copy(hlo_ref.at[src_v.at[g]], ins[b], gsems[b])

        @pl.when(c == 1)
        def _():
            pltpu.async_copy(hhi_ref.at[src_v.at[g]], ins[b], gsems[b])

    def _build(g, b):
        def _bk(k, carry):
            plsc.store_scatter(
                exbs[b], [k * 16 + iota16, dcol_sv[g, pl.ds(k * 16, 16)]],
                w_sv[g, pl.ds(k * 16, 16)])
            return carry
        lax.fori_loop(0, K // 16, _bk, 0)

    def _clean(g, b):
        def _ck(k, carry):
            plsc.store_scatter(
                exbs[b], [k * 16 + iota16, dcol_sv[g, pl.ds(k * 16, 16)]],
                z16)
            return carry
        lax.fori_loop(0, K // 16, _ck, 0)

    def _scale(g, b):
        def _sr(r, carry):
            bc = plsc.load_gather(
                w_sv, [jnp.full((16,), g, i32), jnp.full((16,), r, i32)])
            for j in range(CF // 16):
                outs[b][r, pl.ds(j * 16, 16)] = (
                    ins[b][r, pl.ds(j * 16, 16)] * bc)
            return carry
        lax.fori_loop(0, K, _sr, 0)

    def _sstart(g, b):
        pltpu.async_copy(outs[b], acc_s.at[dst_v.at[g]], ssems[b], add=True)

        @pl.when(c == 0)
        def _():
            pltpu.async_copy(exbs[b], acc_s.at[drow_sv.at[g]], ssems[b],
                             add=True)

    def _sdrain(b):
        _drain(ssems[b], outs[b])

        @pl.when(c == 0)
        def _():
            _drain(ssems[b], outs[b])

    def _slab(sl, carry):
        pltpu.sync_copy(src_ref.at[s, pl.ds(sl * GS, GS)], src_v)
        pltpu.sync_copy(dst_ref.at[s, pl.ds(sl * GS, GS)], dst_v)
        pltpu.sync_copy(ae_ref.at[s, pl.ds(sl * GS, GS)], ae_v)

        def _alpha_g(g2, carry2):
            def _alpha(k, carry3):
                si = src_v[g2, pl.ds(k * 16, 16)]
                di = dst_v[g2, pl.ds(k * 16, 16)]
                al = (plsc.load_gather(asrc_v, [si])
                      + plsc.load_gather(adst_v, [di])
                      + ae_v[g2, pl.ds(k * 16, 16)])
                al = jnp.where(al >= 0, al, 0.2 * al)
                w_sv[g2, pl.ds(k * 16, 16)] = jnp.exp(al)
                dcol_sv[g2, pl.ds(k * 16, 16)] = lax.bitwise_and(di, 63)
                drow_sv[g2, pl.ds(k * 16, 16)] = (
                    DB + lax.shift_right_logical(di, 6))
                return carry3
            lax.fori_loop(0, K // 16, _alpha, 0)
            return carry2
        lax.fori_loop(0, GS, _alpha_g, 0)

        _gstart(0, 0)
        _gstart(1, 1)
        for b in (0, 1):
            g0 = jnp.int32(b)
            _drain(gsems[b], ins[b])
            _scale(g0, b)
            _gstart(g0 + 2, b)

            @pl.when(c == 0)
            def _():
                _build(g0, b)
            _sstart(g0, b)

        def _steady(i, carry2):
            for b in (0, 1):
                g = 2 * i + b
                _sdrain(b)
                _drain(gsems[b], ins[b])
                _scale(g, b)

                @pl.when(g + 2 < GS)
                def _():
                    _gstart(g + 2, b)

                @pl.when(c == 0)
                def _():
                    _clean(g - 2, b)
                    _build(g, b)
                _sstart(g, b)
            return carry2
        lax.fori_loop(1, 15, _steady, 0)

        g30 = jnp.int32(30)
        _sdrain(0)
        _drain(gsems[0], ins[0])
        _scale(g30, 0)

        @pl.when(c == 0)
        def _():
            _clean(g30 - 2, 0)
            _build(g30, 0)
        _sstart(g30, 0)

        _sdrain(0)
        _sdrain(1)

        @pl.when(c == 0)
        def _():
            _clean(jnp.int32(29), 1)
            _clean(g30, 0)
        return carry
    lax.fori_loop(0, NSLAB, _slab, 0)

    plsc.subcore_barrier()
    for t in range(5):
        pltpu.sync_copy(acc_s.at[pl.ds(base + t * K, K)],
                        outp_ref.at[c, pl.ds(base + t * K, K)])
    pltpu.sync_copy(acc_s.at[pl.ds(base + 5 * K, SL - 5 * K)],
                    outp_ref.at[c, pl.ds(base + 5 * K, SL - 5 * K)])


_sc_layer = pl.kernel(
    _sc_body,
    out_type=jax.ShapeDtypeStruct((NC, NSEG, CF), f32),
    mesh=plsc.VectorSubcoreMesh(core_axis_name="c", subcore_axis_name="s"),
    compiler_params=pltpu.CompilerParams(needs_layout_passes=False,
                                         use_tc_tiling_on_sc=False),
    scratch_types=[
        pltpu.VMEM((NP,), f32),
        pltpu.VMEM((NP,), f32),
        pltpu.VMEM((GS, K), i32),
        pltpu.VMEM((GS, K), i32),
        pltpu.VMEM((GS, K), f32),
        pltpu.VMEM((GS, K), f32),
        pltpu.VMEM((GS, K), i32),
        pltpu.VMEM((GS, K), i32),
        pltpu.VMEM((K, CF), f32),
        pltpu.VMEM((K, CF), f32),
        pltpu.VMEM((K, CF), f32),
        pltpu.VMEM((K, CF), f32),
        pltpu.VMEM((K, CF), f32),
        pltpu.VMEM((K, CF), f32),
        pltpu.VMEM_SHARED((NSEG, CF), f32),
        pltpu.SemaphoreType.DMA,
        pltpu.SemaphoreType.DMA,
        pltpu.SemaphoreType.DMA,
        pltpu.SemaphoreType.DMA,
    ],
)


@jax.jit
def kernel(x, edge_index, edge_attr, W1, We1, as1, ad1, ae1, b1,
           W2, We2, as2, ad2, ae2, b2, Wf, bf):
    loop = jnp.arange(N, dtype=i32)
    src = jnp.concatenate([edge_index[0].astype(i32), loop,
                           jnp.zeros((EP - ETOT,), i32)])
    dst = jnp.concatenate([edge_index[1].astype(i32), loop,
                           jnp.full((EP - ETOT,), N, i32)])
    src3 = src.reshape(NS, G, K)
    dst3 = dst.reshape(NS, G, K)
    eat = edge_attr.T

    hlo, hhi, asrc1, adst1, ae1p, ae2p = _tc1(x, W1, as1, ad1, eat, We1, ae1,
                                              We2, ae2)
    p1 = _sc_layer(src3, dst3, ae1p.reshape(NS, G, K), asrc1, adst1,
                   hlo, hhi)
    den1 = p1[0, DB:DB + DR, :].reshape(DR * CF)
    hlo2, hhi2, asrc2, adst2 = _tc2(p1, den1, b1, W2, as2, ad2)
    p2 = _sc_layer(src3, dst3, ae2p.reshape(NS, G, K), asrc2, adst2,
                   hlo2, hhi2)
    den2 = p2[0, DB:DB + DR, :].reshape(DR * CF)
    return _tc3(p2, den2, b2, Wf, bf)


# speedup vs baseline: 34.9541x; 1.8626x over previous
import jax
import jax.numpy as jnp
from jax import lax
from jax.experimental import pallas as pl
from jax.experimental.pallas import tpu as pltpu
from jax.experimental.pallas import tpu_sc as plsc

N = 10000
E = 320000
ETOT = E + N
C = 128
A = 16
CF = C // 2
NP = N + 16
DB = N + 16
DR = 157
NSEG = 10176
NC = 2
NS = 16
K = 112
G = 186
CH = G * K
EP = NS * CH
SL = NSEG // NS
GS = 31
NSLAB = G // GS

f32 = jnp.float32
i32 = jnp.int32


def _tc1_body(x_ref, W1_ref, as1_ref, ad1_ref, eat_ref, We1_ref, ae1_ref,
              We2_ref, ae2_ref,
              hlo_ref, hhi_ref, asrc_ref, adst_ref, ae1p_ref, ae2p_ref):
    h = jnp.dot(x_ref[...], W1_ref[...], preferred_element_type=f32)
    hlo_ref[...] = h[:, :CF]
    hhi_ref[...] = h[:, CF:]
    z16 = jnp.zeros((16,), f32)
    asrc_ref[...] = jnp.concatenate([jnp.dot(h, as1_ref[...]), z16])
    adst_ref[...] = jnp.concatenate([jnp.dot(h, ad1_ref[...]), z16])
    for We_ref, ae_ref, out_ref in ((We1_ref, ae1_ref, ae1p_ref),
                                    (We2_ref, ae2_ref, ae2p_ref)):
        v0 = jnp.sum(We_ref[0, :] * ae_ref[...])
        v1 = jnp.sum(We_ref[1, :] * ae_ref[...])
        ae_real = eat_ref[0, :] * v0 + eat_ref[1, :] * v1
        loopv = jnp.mean(eat_ref[0, :]) * v0 + jnp.mean(eat_ref[1, :]) * v1
        out_ref[...] = jnp.concatenate(
            [ae_real, jnp.full((EP - E,), loopv, f32)])


_tc1 = pl.pallas_call(
    _tc1_body,
    out_shape=[
        jax.ShapeDtypeStruct((N, CF), f32),
        jax.ShapeDtypeStruct((N, CF), f32),
        jax.ShapeDtypeStruct((NP,), f32),
        jax.ShapeDtypeStruct((NP,), f32),
        jax.ShapeDtypeStruct((EP,), f32),
        jax.ShapeDtypeStruct((EP,), f32),
    ],
)


def _norm_h(p_ref, den0_ref, den1_ref, b_ref):
    acc = jnp.concatenate([p_ref[0, :N, :], p_ref[1, :N, :]], axis=-1)
    den = den0_ref[...][:N] + den1_ref[...][:N]
    return jnp.maximum(acc / jnp.maximum(den, 1e-16)[:, None]
                       + b_ref[...][None, :], 0.0)


def _tc2_body(p_ref, den0_ref, den1_ref, b_ref, W_ref, as_ref, ad_ref,
              hlo_ref, hhi_ref, asrc_ref, adst_ref):
    h = _norm_h(p_ref, den0_ref, den1_ref, b_ref)
    hw = jnp.dot(h, W_ref[...], preferred_element_type=f32)
    hlo_ref[...] = hw[:, :CF]
    hhi_ref[...] = hw[:, CF:]
    z16 = jnp.zeros((16,), f32)
    asrc_ref[...] = jnp.concatenate([jnp.dot(hw, as_ref[...]), z16])
    adst_ref[...] = jnp.concatenate([jnp.dot(hw, ad_ref[...]), z16])


_tc2 = pl.pallas_call(
    _tc2_body,
    out_shape=[
        jax.ShapeDtypeStruct((N, CF), f32),
        jax.ShapeDtypeStruct((N, CF), f32),
        jax.ShapeDtypeStruct((NP,), f32),
        jax.ShapeDtypeStruct((NP,), f32),
    ],
)


def _tc3_body(p_ref, den0_ref, den1_ref, b_ref, Wf_ref, bf_ref, y_ref):
    h = _norm_h(p_ref, den0_ref, den1_ref, b_ref)
    g = jnp.mean(h, axis=0)
    y_ref[...] = jnp.tanh(jnp.dot(g, Wf_ref[...]) + bf_ref[...])


_tc3 = pl.pallas_call(
    _tc3_body,
    out_shape=jax.ShapeDtypeStruct((A,), f32),
)


def _sc_body(src_ref, dst_ref, ae_ref, asrc_ref, adst_ref, hlo_ref, hhi_ref,
             outp_ref,
             asrc_v, adst_v, src_v, dst_v, ae_v, w_sv, drow_sv, dcol_sv,
             in0, in1, out0, out1, exb0, exb1, acc_s,
             gsem0, gsem1, ssem0, ssem1):
    c = lax.axis_index("c")
    s = lax.axis_index("s")
    z16 = jnp.zeros((16,), f32)
    ins = (in0, in1)
    outs = (out0, out1)
    exbs = (exb0, exb1)
    gsems = (gsem0, gsem1)
    ssems = (ssem0, ssem1)

    def _zr(r, carry):
        for j in range(CF // 16):
            out0[r, pl.ds(j * 16, 16)] = z16
            exb0[r, pl.ds(j * 16, 16)] = z16
            exb1[r, pl.ds(j * 16, 16)] = z16
        return carry
    lax.fori_loop(0, K, _zr, 0)
    base = s * SL
    for t in range(5):
        pltpu.sync_copy(out0, acc_s.at[pl.ds(base + t * K, K)])
    pltpu.sync_copy(out0.at[pl.ds(0, SL - 5 * K)],
                    acc_s.at[pl.ds(base + 5 * K, SL - 5 * K)])
    plsc.subcore_barrier()

    pltpu.sync_copy(asrc_ref, asrc_v)
    pltpu.sync_copy(adst_ref, adst_v)

    iota16 = lax.iota(i32, 16)

    def _drain(sem, buf):
        pltpu.make_async_copy(hlo_ref.at[pl.ds(0, K)], buf, sem).wait()

    def _gstart(g, b):
        @pl.when(c == 0)
        def _():
            pltpu.async_copy(hlo_ref.at[src_v.at[g]], ins[b], gsems[b])

        @pl.when(c == 1)
        def _():
            pltpu.async_copy(hhi_ref.at[src_v.at[g]], ins[b], gsems[b])

    def _build(g, b):
        for k in range(K // 16):
            plsc.store_scatter(
                exbs[b], [k * 16 + iota16, dcol_sv[g, pl.ds(k * 16, 16)]],
                w_sv[g, pl.ds(k * 16, 16)])

    def _clean(g, b):
        for k in range(K // 16):
            plsc.store_scatter(
                exbs[b], [k * 16 + iota16, dcol_sv[g, pl.ds(k * 16, 16)]],
                z16)

    def _scale(g, b):
        @plsc.parallel_loop(0, K, step=1, unroll=4)
        def _sr(r):
            bc = plsc.load_gather(
                w_sv, [jnp.full((16,), g, i32), jnp.full((16,), r, i32)])
            for j in range(CF // 16):
                outs[b][r, pl.ds(j * 16, 16)] = (
                    ins[b][r, pl.ds(j * 16, 16)] * bc)

    def _sstart(g, b):
        pltpu.async_copy(outs[b], acc_s.at[dst_v.at[g]], ssems[b], add=True)

        @pl.when(c == b)
        def _():
            pltpu.async_copy(exbs[b], acc_s.at[drow_sv.at[g]], ssems[b],
                             add=True)

    def _sdrain(b):
        _drain(ssems[b], outs[b])

        @pl.when(c == b)
        def _():
            _drain(ssems[b], outs[b])

    def _slab(sl, carry):
        pltpu.sync_copy(src_ref.at[s, pl.ds(sl * GS, GS)], src_v)
        pltpu.sync_copy(dst_ref.at[s, pl.ds(sl * GS, GS)], dst_v)
        pltpu.sync_copy(ae_ref.at[s, pl.ds(sl * GS, GS)], ae_v)

        @plsc.parallel_loop(0, GS, step=1, unroll=1)
        def _alpha(g2):
            for k in range(K // 16):
                si = src_v[g2, pl.ds(k * 16, 16)]
                di = dst_v[g2, pl.ds(k * 16, 16)]
                al = (plsc.load_gather(asrc_v, [si])
                      + plsc.load_gather(adst_v, [di])
                      + ae_v[g2, pl.ds(k * 16, 16)])
                al = jnp.where(al >= 0, al, 0.2 * al)
                w_sv[g2, pl.ds(k * 16, 16)] = jnp.exp(al)
                dcol_sv[g2, pl.ds(k * 16, 16)] = lax.bitwise_and(di, 63)
                drow_sv[g2, pl.ds(k * 16, 16)] = (
                    DB + lax.shift_right_logical(di, 6))

        _gstart(0, 0)
        _gstart(1, 1)
        for b in (0, 1):
            g0 = jnp.int32(b)
            _drain(gsems[b], ins[b])
            _scale(g0, b)
            _gstart(g0 + 2, b)

            @pl.when(c == b)
            def _():
                _build(g0, b)
            _sstart(g0, b)

        def _steady(i, carry2):
            for b in (0, 1):
                g = 2 * i + b
                _sdrain(b)
                _drain(gsems[b], ins[b])
                _scale(g, b)

                @pl.when(g + 2 < GS)
                def _():
                    _gstart(g + 2, b)

                @pl.when(c == b)
                def _():
                    _clean(g - 2, b)
                    _build(g, b)
                _sstart(g, b)
            return carry2
        lax.fori_loop(1, 15, _steady, 0)

        g30 = jnp.int32(30)
        _sdrain(0)
        _drain(gsems[0], ins[0])
        _scale(g30, 0)

        @pl.when(c == 0)
        def _():
            _clean(g30 - 2, 0)
            _build(g30, 0)
        _sstart(g30, 0)

        _sdrain(0)
        _sdrain(1)

        @pl.when(c == 0)
        def _():
            _clean(g30, 0)

        @pl.when(c == 1)
        def _():
            _clean(jnp.int32(29), 1)
        return carry
    lax.fori_loop(0, NSLAB, _slab, 0)

    plsc.subcore_barrier()
    for t in range(5):
        pltpu.sync_copy(acc_s.at[pl.ds(base + t * K, K)],
                        outp_ref.at[c, pl.ds(base + t * K, K)])
    pltpu.sync_copy(acc_s.at[pl.ds(base + 5 * K, SL - 5 * K)],
                    outp_ref.at[c, pl.ds(base + 5 * K, SL - 5 * K)])


_sc_layer = pl.kernel(
    _sc_body,
    out_type=jax.ShapeDtypeStruct((NC, NSEG, CF), f32),
    mesh=plsc.VectorSubcoreMesh(core_axis_name="c", subcore_axis_name="s"),
    compiler_params=pltpu.CompilerParams(needs_layout_passes=False,
                                         use_tc_tiling_on_sc=False),
    scratch_types=[
        pltpu.VMEM((NP,), f32),
        pltpu.VMEM((NP,), f32),
        pltpu.VMEM((GS, K), i32),
        pltpu.VMEM((GS, K), i32),
        pltpu.VMEM((GS, K), f32),
        pltpu.VMEM((GS, K), f32),
        pltpu.VMEM((GS, K), i32),
        pltpu.VMEM((GS, K), i32),
        pltpu.VMEM((K, CF), f32),
        pltpu.VMEM((K, CF), f32),
        pltpu.VMEM((K, CF), f32),
        pltpu.VMEM((K, CF), f32),
        pltpu.VMEM((K, CF), f32),
        pltpu.VMEM((K, CF), f32),
        pltpu.VMEM_SHARED((NSEG, CF), f32),
        pltpu.SemaphoreType.DMA,
        pltpu.SemaphoreType.DMA,
        pltpu.SemaphoreType.DMA,
        pltpu.SemaphoreType.DMA,
    ],
)


@jax.jit
def kernel(x, edge_index, edge_attr, W1, We1, as1, ad1, ae1, b1,
           W2, We2, as2, ad2, ae2, b2, Wf, bf):
    loop = jnp.arange(N, dtype=i32)
    src = jnp.concatenate([edge_index[0].astype(i32), loop,
                           jnp.zeros((EP - ETOT,), i32)])
    dst = jnp.concatenate([edge_index[1].astype(i32), loop,
                           jnp.full((EP - ETOT,), N, i32)])
    src3 = src.reshape(NS, G, K)
    dst3 = dst.reshape(NS, G, K)
    eat = edge_attr.T

    hlo, hhi, asrc1, adst1, ae1p, ae2p = _tc1(x, W1, as1, ad1, eat, We1, ae1,
                                              We2, ae2)
    p1 = _sc_layer(src3, dst3, ae1p.reshape(NS, G, K), asrc1, adst1,
                   hlo, hhi)
    den1a = p1[0, DB:DB + DR, :].reshape(DR * CF)
    den1b = p1[1, DB:DB + DR, :].reshape(DR * CF)
    hlo2, hhi2, asrc2, adst2 = _tc2(p1, den1a, den1b, b1, W2, as2, ad2)
    p2 = _sc_layer(src3, dst3, ae2p.reshape(NS, G, K), asrc2, adst2,
                   hlo2, hhi2)
    den2a = p2[0, DB:DB + DR, :].reshape(DR * CF)
    den2b = p2[1, DB:DB + DR, :].reshape(DR * CF)
    return _tc3(p2, den2a, den2b, b2, Wf, bf)


# speedup vs baseline: 35.9881x; 1.0296x over previous
import jax
import jax.numpy as jnp
from jax import lax
from jax.experimental import pallas as pl
from jax.experimental.pallas import tpu as pltpu
from jax.experimental.pallas import tpu_sc as plsc

N = 10000
E = 320000
ETOT = E + N
C = 128
A = 16
CF = C // 2
NP = N + 16
DB = N + 16
DR = 157
NSEG = 10176
NC = 2
NS = 16
K = 112
G = 186
CH = G * K
EP = NS * CH
SL = NSEG // NS
GS = 31
NSLAB = G // GS

f32 = jnp.float32
i32 = jnp.int32


def _tc1_body(x_ref, W1_ref, as1_ref, ad1_ref, eat_ref, We1_ref, ae1_ref,
              We2_ref, ae2_ref,
              hlo_ref, hhi_ref, asrc_ref, adst_ref, ae1p_ref, ae2p_ref):
    h = jnp.dot(x_ref[...], W1_ref[...], preferred_element_type=f32)
    hlo_ref[...] = h[:, :CF]
    hhi_ref[...] = h[:, CF:]
    z16 = jnp.zeros((16,), f32)
    asrc_ref[...] = jnp.concatenate([jnp.dot(h, as1_ref[...]), z16])
    adst_ref[...] = jnp.concatenate([jnp.dot(h, ad1_ref[...]), z16])
    for We_ref, ae_ref, out_ref in ((We1_ref, ae1_ref, ae1p_ref),
                                    (We2_ref, ae2_ref, ae2p_ref)):
        v0 = jnp.sum(We_ref[0, :] * ae_ref[...])
        v1 = jnp.sum(We_ref[1, :] * ae_ref[...])
        ae_real = eat_ref[0, :] * v0 + eat_ref[1, :] * v1
        loopv = jnp.mean(eat_ref[0, :]) * v0 + jnp.mean(eat_ref[1, :]) * v1
        out_ref[...] = jnp.concatenate(
            [ae_real, jnp.full((EP - E,), loopv, f32)])


_tc1 = pl.pallas_call(
    _tc1_body,
    out_shape=[
        jax.ShapeDtypeStruct((N, CF), f32),
        jax.ShapeDtypeStruct((N, CF), f32),
        jax.ShapeDtypeStruct((NP,), f32),
        jax.ShapeDtypeStruct((NP,), f32),
        jax.ShapeDtypeStruct((EP,), f32),
        jax.ShapeDtypeStruct((EP,), f32),
    ],
)


def _norm_h(p_ref, den0_ref, den1_ref, b_ref):
    acc = jnp.concatenate([p_ref[0, :N, :], p_ref[1, :N, :]], axis=-1)
    den = den0_ref[...][:N] + den1_ref[...][:N]
    return jnp.maximum(acc / jnp.maximum(den, 1e-16)[:, None]
                       + b_ref[...][None, :], 0.0)


def _tc2_body(p_ref, den0_ref, den1_ref, b_ref, W_ref, as_ref, ad_ref,
              hlo_ref, hhi_ref, asrc_ref, adst_ref):
    h = _norm_h(p_ref, den0_ref, den1_ref, b_ref)
    hw = jnp.dot(h, W_ref[...], preferred_element_type=f32)
    hlo_ref[...] = hw[:, :CF]
    hhi_ref[...] = hw[:, CF:]
    z16 = jnp.zeros((16,), f32)
    asrc_ref[...] = jnp.concatenate([jnp.dot(hw, as_ref[...]), z16])
    adst_ref[...] = jnp.concatenate([jnp.dot(hw, ad_ref[...]), z16])


_tc2 = pl.pallas_call(
    _tc2_body,
    out_shape=[
        jax.ShapeDtypeStruct((N, CF), f32),
        jax.ShapeDtypeStruct((N, CF), f32),
        jax.ShapeDtypeStruct((NP,), f32),
        jax.ShapeDtypeStruct((NP,), f32),
    ],
)


def _tc3_body(p_ref, den0_ref, den1_ref, b_ref, Wf_ref, bf_ref, y_ref):
    h = _norm_h(p_ref, den0_ref, den1_ref, b_ref)
    g = jnp.mean(h, axis=0)
    y_ref[...] = jnp.tanh(jnp.dot(g, Wf_ref[...]) + bf_ref[...])


_tc3 = pl.pallas_call(
    _tc3_body,
    out_shape=jax.ShapeDtypeStruct((A,), f32),
)


def _sc_body(src_ref, dst_ref, ae_ref, asrc_ref, adst_ref, hlo_ref, hhi_ref,
             outp_ref,
             asrc_v, adst_v, src_v, dst_v, ae_v, w_sv, drow_sv, dcol_sv,
             in0, in1, out0, out1, exb0, exb1, acc_s,
             gsem0, gsem1, ssem0, ssem1):
    c = lax.axis_index("c")
    s = lax.axis_index("s")
    z16 = jnp.zeros((16,), f32)
    ins = (in0, in1)
    outs = (out0, out1)
    exbs = (exb0, exb1)
    gsems = (gsem0, gsem1)
    ssems = (ssem0, ssem1)

    def _zr(r, carry):
        for j in range(CF // 16):
            out0[r, pl.ds(j * 16, 16)] = z16
            exb0[r, pl.ds(j * 16, 16)] = z16
            exb1[r, pl.ds(j * 16, 16)] = z16
        return carry
    lax.fori_loop(0, K, _zr, 0)
    base = s * SL
    for t in range(5):
        pltpu.sync_copy(out0, acc_s.at[pl.ds(base + t * K, K)])
    pltpu.sync_copy(out0.at[pl.ds(0, SL - 5 * K)],
                    acc_s.at[pl.ds(base + 5 * K, SL - 5 * K)])
    plsc.subcore_barrier()

    pltpu.sync_copy(asrc_ref, asrc_v)
    pltpu.sync_copy(adst_ref, adst_v)

    iota16 = lax.iota(i32, 16)

    def _drain(sem, buf):
        pltpu.make_async_copy(hlo_ref.at[pl.ds(0, K)], buf, sem).wait()

    def _gstart(g, b):
        @pl.when(c == 0)
        def _():
            pltpu.async_copy(hlo_ref.at[src_v.at[g]], ins[b], gsems[b])

        @pl.when(c == 1)
        def _():
            pltpu.async_copy(hhi_ref.at[src_v.at[g]], ins[b], gsems[b])

    def _build(g, b):
        for k in range(K // 16):
            plsc.store_scatter(
                exbs[b], [k * 16 + iota16, dcol_sv[g, pl.ds(k * 16, 16)]],
                w_sv[g, pl.ds(k * 16, 16)])

    def _clean(g, b):
        for k in range(K // 16):
            plsc.store_scatter(
                exbs[b], [k * 16 + iota16, dcol_sv[g, pl.ds(k * 16, 16)]],
                z16)

    def _scale(g, b):
        @plsc.parallel_loop(0, K // 16, step=1, unroll=1)
        def _sr(r16):
            r0 = r16 * 16
            wrow = w_sv[g, pl.ds(r0, 16)]
            for i in range(16):
                bc = jnp.full((16,), wrow[i], f32)
                for j in range(CF // 16):
                    outs[b][r0 + i, pl.ds(j * 16, 16)] = (
                        ins[b][r0 + i, pl.ds(j * 16, 16)] * bc)

    def _sstart(g, b):
        pltpu.async_copy(outs[b], acc_s.at[dst_v.at[g]], ssems[b], add=True)

        @pl.when(c == b)
        def _():
            pltpu.async_copy(exbs[b], acc_s.at[drow_sv.at[g]], ssems[b],
                             add=True)

    def _sdrain(b):
        _drain(ssems[b], outs[b])

        @pl.when(c == b)
        def _():
            _drain(ssems[b], outs[b])

    def _slab(sl, carry):
        pltpu.sync_copy(src_ref.at[s, pl.ds(sl * GS, GS)], src_v)
        pltpu.sync_copy(dst_ref.at[s, pl.ds(sl * GS, GS)], dst_v)
        pltpu.sync_copy(ae_ref.at[s, pl.ds(sl * GS, GS)], ae_v)

        @plsc.parallel_loop(0, GS, step=1, unroll=1)
        def _alpha(g2):
            for k in range(K // 16):
                si = src_v[g2, pl.ds(k * 16, 16)]
                di = dst_v[g2, pl.ds(k * 16, 16)]
                al = (plsc.load_gather(asrc_v, [si])
                      + plsc.load_gather(adst_v, [di])
                      + ae_v[g2, pl.ds(k * 16, 16)])
                al = jnp.where(al >= 0, al, 0.2 * al)
                w_sv[g2, pl.ds(k * 16, 16)] = jnp.exp(al)
                dcol_sv[g2, pl.ds(k * 16, 16)] = lax.bitwise_and(di, 63)
                drow_sv[g2, pl.ds(k * 16, 16)] = (
                    DB + lax.shift_right_logical(di, 6))

        _gstart(0, 0)
        _gstart(1, 1)
        for b in (0, 1):
            g0 = jnp.int32(b)
            _drain(gsems[b], ins[b])
            _scale(g0, b)
            _gstart(g0 + 2, b)

            @pl.when(c == b)
            def _():
                _build(g0, b)
            _sstart(g0, b)

        def _steady(i, carry2):
            for b in (0, 1):
                g = 2 * i + b
                _sdrain(b)
                _drain(gsems[b], ins[b])
                _scale(g, b)

                @pl.when(g + 2 < GS)
                def _():
                    _gstart(g + 2, b)

                @pl.when(c == b)
                def _():
                    _clean(g - 2, b)
                    _build(g, b)
                _sstart(g, b)
            return carry2
        lax.fori_loop(1, 15, _steady, 0)

        g30 = jnp.int32(30)
        _sdrain(0)
        _drain(gsems[0], ins[0])
        _scale(g30, 0)

        @pl.when(c == 0)
        def _():
            _clean(g30 - 2, 0)
            _build(g30, 0)
        _sstart(g30, 0)

        _sdrain(0)
        _sdrain(1)

        @pl.when(c == 0)
        def _():
            _clean(g30, 0)

        @pl.when(c == 1)
        def _():
            _clean(jnp.int32(29), 1)
        return carry
    lax.fori_loop(0, NSLAB, _slab, 0)

    plsc.subcore_barrier()
    for t in range(5):
        pltpu.sync_copy(acc_s.at[pl.ds(base + t * K, K)],
                        outp_ref.at[c, pl.ds(base + t * K, K)])
    pltpu.sync_copy(acc_s.at[pl.ds(base + 5 * K, SL - 5 * K)],
                    outp_ref.at[c, pl.ds(base + 5 * K, SL - 5 * K)])


_sc_layer = pl.kernel(
    _sc_body,
    out_type=jax.ShapeDtypeStruct((NC, NSEG, CF), f32),
    mesh=plsc.VectorSubcoreMesh(core_axis_name="c", subcore_axis_name="s"),
    compiler_params=pltpu.CompilerParams(needs_layout_passes=False,
                                         use_tc_tiling_on_sc=False),
    scratch_types=[
        pltpu.VMEM((NP,), f32),
        pltpu.VMEM((NP,), f32),
        pltpu.VMEM((GS, K), i32),
        pltpu.VMEM((GS, K), i32),
        pltpu.VMEM((GS, K), f32),
        pltpu.VMEM((GS, K), f32),
        pltpu.VMEM((GS, K), i32),
        pltpu.VMEM((GS, K), i32),
        pltpu.VMEM((K, CF), f32),
        pltpu.VMEM((K, CF), f32),
        pltpu.VMEM((K, CF), f32),
        pltpu.VMEM((K, CF), f32),
        pltpu.VMEM((K, CF), f32),
        pltpu.VMEM((K, CF), f32),
        pltpu.VMEM_SHARED((NSEG, CF), f32),
        pltpu.SemaphoreType.DMA,
        pltpu.SemaphoreType.DMA,
        pltpu.SemaphoreType.DMA,
        pltpu.SemaphoreType.DMA,
    ],
)


@jax.jit
def kernel(x, edge_index, edge_attr, W1, We1, as1, ad1, ae1, b1,
           W2, We2, as2, ad2, ae2, b2, Wf, bf):
    loop = jnp.arange(N, dtype=i32)
    src = jnp.concatenate([edge_index[0].astype(i32), loop,
                           jnp.zeros((EP - ETOT,), i32)])
    dst = jnp.concatenate([edge_index[1].astype(i32), loop,
                           jnp.full((EP - ETOT,), N, i32)])
    src3 = src.reshape(NS, G, K)
    dst3 = dst.reshape(NS, G, K)
    eat = edge_attr.T

    hlo, hhi, asrc1, adst1, ae1p, ae2p = _tc1(x, W1, as1, ad1, eat, We1, ae1,
                                              We2, ae2)
    p1 = _sc_layer(src3, dst3, ae1p.reshape(NS, G, K), asrc1, adst1,
                   hlo, hhi)
    den1a = p1[0, DB:DB + DR, :].reshape(DR * CF)
    den1b = p1[1, DB:DB + DR, :].reshape(DR * CF)
    hlo2, hhi2, asrc2, adst2 = _tc2(p1, den1a, den1b, b1, W2, as2, ad2)
    p2 = _sc_layer(src3, dst3, ae2p.reshape(NS, G, K), asrc2, adst2,
                   hlo2, hhi2)
    den2a = p2[0, DB:DB + DR, :].reshape(DR * CF)
    den2b = p2[1, DB:DB + DR, :].reshape(DR * CF)
    return _tc3(p2, den2a, den2b, b2, Wf, bf)


# speedup vs baseline: 41.7372x; 1.1597x over previous
import jax
import jax.numpy as jnp
from jax import lax
from jax.experimental import pallas as pl
from jax.experimental.pallas import tpu as pltpu
from jax.experimental.pallas import tpu_sc as plsc

N = 10000
E = 320000
ETOT = E + N
C = 128
A = 16
CF = C // 2
NP = N + 16
NSEG = 10240
NC = 2
NS = 16
K = 112
G = 186
CH = G * K
EP = NS * CH
SL = NSEG // NS
GS = 31
NSLAB = G // GS

f32 = jnp.float32
i32 = jnp.int32


def _tc1_body(x_ref, W1_ref, as1_ref, ad1_ref, eat_ref, We1_ref, ae1_ref,
              We2_ref, ae2_ref, ei_ref,
              hlo_ref, hhi_ref, asrc_ref, adst_ref, ae1p_ref, ae2p_ref,
              src_ref, dst_ref):
    loop = lax.iota(i32, N)
    src_ref[...] = jnp.concatenate(
        [ei_ref[0, :], loop, jnp.zeros((EP - ETOT,), i32)])
    dst_ref[...] = jnp.concatenate(
        [ei_ref[1, :], loop, jnp.full((EP - ETOT,), N, i32)])
    h = jnp.dot(x_ref[...], W1_ref[...], preferred_element_type=f32)
    hlo_ref[...] = h[:, :CF]
    hhi_ref[...] = h[:, CF:]
    z16 = jnp.zeros((16,), f32)
    asrc_ref[...] = jnp.concatenate([jnp.dot(h, as1_ref[...]), z16])
    adst_ref[...] = jnp.concatenate([jnp.dot(h, ad1_ref[...]), z16])
    for We_ref, ae_ref, out_ref in ((We1_ref, ae1_ref, ae1p_ref),
                                    (We2_ref, ae2_ref, ae2p_ref)):
        v0 = jnp.sum(We_ref[0, :] * ae_ref[...])
        v1 = jnp.sum(We_ref[1, :] * ae_ref[...])
        ae_real = eat_ref[0, :] * v0 + eat_ref[1, :] * v1
        loopv = jnp.mean(eat_ref[0, :]) * v0 + jnp.mean(eat_ref[1, :]) * v1
        out_ref[...] = jnp.concatenate(
            [ae_real, jnp.full((EP - E,), loopv, f32)])


_tc1 = pl.pallas_call(
    _tc1_body,
    out_shape=[
        jax.ShapeDtypeStruct((N, CF), f32),
        jax.ShapeDtypeStruct((N, CF), f32),
        jax.ShapeDtypeStruct((NP,), f32),
        jax.ShapeDtypeStruct((NP,), f32),
        jax.ShapeDtypeStruct((EP,), f32),
        jax.ShapeDtypeStruct((EP,), f32),
        jax.ShapeDtypeStruct((EP,), i32),
        jax.ShapeDtypeStruct((EP,), i32),
    ],
)


def _norm_h(p_ref, den_ref, b_ref):
    acc = jnp.concatenate([p_ref[0, :N, :], p_ref[1, :N, :]], axis=-1)
    den = den_ref[0, :N] + den_ref[1, :N]
    return jnp.maximum(acc / jnp.maximum(den, 1e-16)[:, None]
                       + b_ref[...][None, :], 0.0)


def _tc2_body(p_ref, den_ref, b_ref, W_ref, as_ref, ad_ref,
              hlo_ref, hhi_ref, asrc_ref, adst_ref):
    h = _norm_h(p_ref, den_ref, b_ref)
    hw = jnp.dot(h, W_ref[...], preferred_element_type=f32)
    hlo_ref[...] = hw[:, :CF]
    hhi_ref[...] = hw[:, CF:]
    z16 = jnp.zeros((16,), f32)
    asrc_ref[...] = jnp.concatenate([jnp.dot(hw, as_ref[...]), z16])
    adst_ref[...] = jnp.concatenate([jnp.dot(hw, ad_ref[...]), z16])


_tc2 = pl.pallas_call(
    _tc2_body,
    out_shape=[
        jax.ShapeDtypeStruct((N, CF), f32),
        jax.ShapeDtypeStruct((N, CF), f32),
        jax.ShapeDtypeStruct((NP,), f32),
        jax.ShapeDtypeStruct((NP,), f32),
    ],
)


def _tc3_body(p_ref, den_ref, b_ref, Wf_ref, bf_ref, y_ref):
    h = _norm_h(p_ref, den_ref, b_ref)
    g = jnp.mean(h, axis=0)
    y_ref[...] = jnp.tanh(jnp.dot(g, Wf_ref[...]) + bf_ref[...])


_tc3 = pl.pallas_call(
    _tc3_body,
    out_shape=jax.ShapeDtypeStruct((A,), f32),
)


def _sc_body(src_ref, dst_ref, ae_ref, asrc_ref, adst_ref, hlo_ref, hhi_ref,
             outp_ref, outd_ref,
             asrc_v, adst_v, src_v, dst_v, ae_v, w_sv, zden_v,
             in0, in1, out0, out1, acc_s, dacc_s,
             gsem0, gsem1, ssem0, ssem1):
    c = lax.axis_index("c")
    s = lax.axis_index("s")
    z16 = jnp.zeros((16,), f32)
    ins = (in0, in1)
    outs = (out0, out1)
    gsems = (gsem0, gsem1)
    ssems = (ssem0, ssem1)

    def _zr(r, carry):
        for j in range(CF // 16):
            out0[r, pl.ds(j * 16, 16)] = z16
        return carry
    lax.fori_loop(0, K, _zr, 0)
    for t in range(SL // 16):
        zden_v[pl.ds(t * 16, 16)] = z16
    base = s * SL
    for t in range(5):
        pltpu.sync_copy(out0, acc_s.at[pl.ds(base + t * K, K)])
    pltpu.sync_copy(out0.at[pl.ds(0, SL - 5 * K)],
                    acc_s.at[pl.ds(base + 5 * K, SL - 5 * K)])
    pltpu.sync_copy(zden_v, dacc_s.at[pl.ds(base, SL)])
    plsc.subcore_barrier()

    pltpu.sync_copy(asrc_ref, asrc_v)
    pltpu.sync_copy(adst_ref, adst_v)

    def _drain_rows(sem, buf):
        pltpu.make_async_copy(hlo_ref.at[pl.ds(0, K)], buf, sem).wait()

    def _drain_den(sem):
        pltpu.make_async_copy(ae_ref.at[s, 0], zden_v.at[pl.ds(0, K)],
                              sem).wait()

    def _gstart(g, b):
        @pl.when(c == 0)
        def _():
            pltpu.async_copy(hlo_ref.at[src_v.at[g]], ins[b], gsems[b])

        @pl.when(c == 1)
        def _():
            pltpu.async_copy(hhi_ref.at[src_v.at[g]], ins[b], gsems[b])

    def _scale(g, b):
        @plsc.parallel_loop(0, K // 16, step=1, unroll=1)
        def _sr(r16):
            r0 = r16 * 16
            wrow = w_sv[g, pl.ds(r0, 16)]
            for i in range(16):
                bc = jnp.full((16,), wrow[i], f32)
                for j in range(CF // 16):
                    outs[b][r0 + i, pl.ds(j * 16, 16)] = (
                        ins[b][r0 + i, pl.ds(j * 16, 16)] * bc)

    def _sstart(g, b):
        pltpu.async_copy(outs[b], acc_s.at[dst_v.at[g]], ssems[b], add=True)

        @pl.when(c == b)
        def _():
            pltpu.async_copy(w_sv.at[g], dacc_s.at[dst_v.at[g]], ssems[b],
                             add=True)

    def _sdrain(b):
        _drain_rows(ssems[b], outs[b])

        @pl.when(c == b)
        def _():
            _drain_den(ssems[b])

    def _slab(sl, carry):
        pltpu.sync_copy(src_ref.at[s, pl.ds(sl * GS, GS)], src_v)
        pltpu.sync_copy(dst_ref.at[s, pl.ds(sl * GS, GS)], dst_v)
        pltpu.sync_copy(ae_ref.at[s, pl.ds(sl * GS, GS)], ae_v)

        @plsc.parallel_loop(0, GS, step=1, unroll=1)
        def _alpha(g2):
            for k in range(K // 16):
                si = src_v[g2, pl.ds(k * 16, 16)]
                di = dst_v[g2, pl.ds(k * 16, 16)]
                al = (plsc.load_gather(asrc_v, [si])
                      + plsc.load_gather(adst_v, [di])
                      + ae_v[g2, pl.ds(k * 16, 16)])
                al = jnp.where(al >= 0, al, 0.2 * al)
                w_sv[g2, pl.ds(k * 16, 16)] = jnp.exp(al)

        _gstart(0, 0)
        _gstart(1, 1)
        for b in (0, 1):
            g0 = jnp.int32(b)
            _drain_rows(gsems[b], ins[b])
            _scale(g0, b)
            _gstart(g0 + 2, b)
            _sstart(g0, b)

        def _steady(i, carry2):
            for b in (0, 1):
                g = 2 * i + b
                _sdrain(b)
                _drain_rows(gsems[b], ins[b])
                _scale(g, b)

                @pl.when(g + 2 < GS)
                def _():
                    _gstart(g + 2, b)
                _sstart(g, b)
            return carry2
        lax.fori_loop(1, 15, _steady, 0)

        g30 = jnp.int32(30)
        _sdrain(0)
        _drain_rows(gsems[0], ins[0])
        _scale(g30, 0)
        _sstart(g30, 0)

        _sdrain(0)
        _sdrain(1)
        return carry
    lax.fori_loop(0, NSLAB, _slab, 0)

    plsc.subcore_barrier()
    for t in range(5):
        pltpu.sync_copy(acc_s.at[pl.ds(base + t * K, K)],
                        outp_ref.at[c, pl.ds(base + t * K, K)])
    pltpu.sync_copy(acc_s.at[pl.ds(base + 5 * K, SL - 5 * K)],
                    outp_ref.at[c, pl.ds(base + 5 * K, SL - 5 * K)])
    pltpu.sync_copy(dacc_s.at[pl.ds(base, SL)],
                    outd_ref.at[c, pl.ds(base, SL)])


_sc_layer = pl.kernel(
    _sc_body,
    out_type=[
        jax.ShapeDtypeStruct((NC, NSEG, CF), f32),
        jax.ShapeDtypeStruct((NC, NSEG), f32),
    ],
    mesh=plsc.VectorSubcoreMesh(core_axis_name="c", subcore_axis_name="s"),
    compiler_params=pltpu.CompilerParams(needs_layout_passes=False,
                                         use_tc_tiling_on_sc=False),
    scratch_types=[
        pltpu.VMEM((NP,), f32),
        pltpu.VMEM((NP,), f32),
        pltpu.VMEM((GS, K), i32),
        pltpu.VMEM((GS, K), i32),
        pltpu.VMEM((GS, K), f32),
        pltpu.VMEM((GS, K), f32),
        pltpu.VMEM((SL,), f32),
        pltpu.VMEM((K, CF), f32),
        pltpu.VMEM((K, CF), f32),
        pltpu.VMEM((K, CF), f32),
        pltpu.VMEM((K, CF), f32),
        pltpu.VMEM_SHARED((NSEG, CF), f32),
        pltpu.VMEM_SHARED((NSEG,), f32),
        pltpu.SemaphoreType.DMA,
        pltpu.SemaphoreType.DMA,
        pltpu.SemaphoreType.DMA,
        pltpu.SemaphoreType.DMA,
    ],
)


@jax.jit
def kernel(x, edge_index, edge_attr, W1, We1, as1, ad1, ae1, b1,
           W2, We2, as2, ad2, ae2, b2, Wf, bf):
    eat = edge_attr.T

    hlo, hhi, asrc1, adst1, ae1p, ae2p, src_p, dst_p = _tc1(
        x, W1, as1, ad1, eat, We1, ae1, We2, ae2, edge_index.astype(i32))
    src3 = src_p.reshape(NS, G, K)
    dst3 = dst_p.reshape(NS, G, K)

    p1, d1 = _sc_layer(src3, dst3, ae1p.reshape(NS, G, K), asrc1, adst1,
                       hlo, hhi)
    hlo2, hhi2, asrc2, adst2 = _tc2(p1, d1, b1, W2, as2, ad2)
    p2, d2 = _sc_layer(src3, dst3, ae2p.reshape(NS, G, K), asrc2, adst2,
                       hlo2, hhi2)
    return _tc3(p2, d2, b2, Wf, bf)
